# Initial kernel scaffold; baseline (speedup 1.0000x reference)
import jax
import jax.numpy as jnp
from jax.experimental import pallas as pl


def kernel(h, edge_index, W1, aL1, aR1, b1, W2, aL2, aR2, b2, Wfc, bfc):
    raise NotImplementedError("write your pallas kernel here")


# speedup vs baseline: 13.2373x; 13.2373x over previous
import functools

import jax
import jax.numpy as jnp
from jax import lax
from jax.experimental import pallas as pl
from jax.experimental.pallas import tpu as pltpu
from jax.experimental.pallas import tpu_sc as plsc

N = 10000
E = 320000
D = 128
DP = 144
NC = 2
NS = 16
NW = NC * NS
EPT = E // NW
K = 80
NCHUNK = EPT // K
NPAD = 10240
STRIPE = NPAD // NS
ZROWS = 128


def _leaky(x):
    return jnp.where(x >= 0.0, x, 0.2 * x)


def _attn_aux(z, aL, aR):
    el = jnp.sum(z * aL[None, :], axis=1)
    er = jnp.sum(z * aR[None, :], axis=1)
    m = _leaky(jnp.max(el) + er)
    return el, er, m


def _prep_body(h_ref, W_ref, aL_ref, aR_ref, z_ref, el_ref, er_ref, m_ref):
    z = jnp.dot(h_ref[...], W_ref[...], preferred_element_type=jnp.float32)
    z_ref[...] = z
    el, er, m = _attn_aux(z, aL_ref[...], aR_ref[...])
    el_ref[...] = el
    er_ref[...] = er
    m_ref[...] = m


_AUX_SHAPES = [
    jax.ShapeDtypeStruct((N, D), jnp.float32),
    jax.ShapeDtypeStruct((N,), jnp.float32),
    jax.ShapeDtypeStruct((N,), jnp.float32),
    jax.ShapeDtypeStruct((N,), jnp.float32),
]


def _prep(h, W, aL, aR):
    return pl.pallas_call(_prep_body, out_shape=_AUX_SHAPES)(h, W, aL, aR)


def _merge(p_ref, b_ref):
    S = p_ref[0, 0:N, 0:D] + p_ref[1, 0:N, 0:D]
    den = p_ref[0, 0:N, D:D + 1] + p_ref[1, 0:N, D:D + 1]
    return jnp.maximum(S / (den + 1e-9) + b_ref[...][None, :], 0.0)


def _merge_prep_body(p_ref, b_ref, W_ref, aL_ref, aR_ref,
                     z_ref, el_ref, er_ref, m_ref):
    hcur = _merge(p_ref, b_ref)
    z = jnp.dot(hcur, W_ref[...], preferred_element_type=jnp.float32)
    z_ref[...] = z
    el, er, m = _attn_aux(z, aL_ref[...], aR_ref[...])
    el_ref[...] = el
    er_ref[...] = er
    m_ref[...] = m


def _merge_prep(p, b, W, aL, aR):
    return pl.pallas_call(_merge_prep_body, out_shape=_AUX_SHAPES)(p, b, W, aL, aR)


def _final_body(p_ref, b_ref, Wfc_ref, bfc_ref, o_ref):
    h2 = _merge(p_ref, b_ref)
    hg = jnp.mean(h2, axis=0, keepdims=True)
    logits = jnp.dot(hg, Wfc_ref[...],
                     preferred_element_type=jnp.float32) + bfc_ref[...][None, :]
    m = jnp.max(logits, axis=1, keepdims=True)
    sh = logits - m
    o_ref[...] = sh - jnp.log(jnp.sum(jnp.exp(sh), axis=1, keepdims=True))


def _final(p, b, Wfc, bfc):
    return pl.pallas_call(
        _final_body,
        out_shape=jax.ShapeDtypeStruct((1, 2), jnp.float32),
    )(p, b, Wfc, bfc)


def _edge_body(z_hbm, el_hbm, er_hbm, m_hbm, src_hbm, dst_hbm, out_hbm,
               src_v, dst_v, rows_v, staged_v, els_v, erd_v, md_v, ee_v,
               acc_sh, sem):
    c = lax.axis_index("c")
    s = lax.axis_index("s")
    wid = c * NS + s

    zero16 = jnp.zeros((16,), jnp.float32)

    def _zrow(i, carry):
        for cc in range(DP // 16):
            staged_v[i, pl.ds(cc * 16, 16)] = zero16
        return carry

    lax.fori_loop(0, K, _zrow, 0)

    def _zcopy(j, carry):
        pltpu.sync_copy(staged_v, acc_sh.at[pl.ds(s * STRIPE + j * K, K)])
        return carry

    lax.fori_loop(0, STRIPE // K, _zcopy, 0)
    plsc.subcore_barrier()

    onehot = jnp.where(lax.iota(jnp.int32, 16) == 0, 1.0, 0.0)
    ebase = wid * EPT

    def _chunk(t, carry):
        base = ebase + t * K
        pltpu.sync_copy(src_hbm.at[pl.ds(base, K)], src_v)
        pltpu.sync_copy(dst_hbm.at[pl.ds(base, K)], dst_v)
        cp_rows = pltpu.async_copy(z_hbm.at[src_v], rows_v, sem)
        cp_els = pltpu.async_copy(el_hbm.at[src_v], els_v, sem)
        cp_erd = pltpu.async_copy(er_hbm.at[dst_v], erd_v, sem)
        cp_md = pltpu.async_copy(m_hbm.at[dst_v], md_v, sem)
        cp_rows.wait()
        cp_els.wait()
        cp_erd.wait()
        cp_md.wait()

        for i in range(K // 16):
            els = els_v[pl.ds(i * 16, 16)]
            erd = erd_v[pl.ds(i * 16, 16)]
            md = md_v[pl.ds(i * 16, 16)]
            ee_v[pl.ds(i * 16, 16)] = jnp.exp(_leaky(els + erd) - md)

        def _row(r, cy):
            eer = ee_v[pl.ds(r, 16)][0]
            for cc in range(D // 16):
                staged_v[r, pl.ds(cc * 16, 16)] = (
                    rows_v[r, pl.ds(cc * 16, 16)] * eer)
            staged_v[r, pl.ds(D, 16)] = onehot * eer
            return cy

        lax.fori_loop(0, K, _row, 0)
        pltpu.sync_copy(staged_v, acc_sh.at[dst_v], add=True)
        return carry

    lax.fori_loop(0, NCHUNK, _chunk, 0)
    plsc.subcore_barrier()

    def _wcopy(j, carry):
        off = s * STRIPE + j * ZROWS
        pltpu.sync_copy(acc_sh.at[pl.ds(off, ZROWS)],
                        out_hbm.at[c, pl.ds(off, ZROWS)])
        return carry

    lax.fori_loop(0, STRIPE // ZROWS, _wcopy, 0)


def _edge_aggregate(z, el, er, m, src, dst):
    mesh = plsc.VectorSubcoreMesh(core_axis_name="c", subcore_axis_name="s")
    kfn = pl.kernel(
        _edge_body,
        out_type=jax.ShapeDtypeStruct((NC, NPAD, DP), jnp.float32),
        mesh=mesh,
        compiler_params=pltpu.CompilerParams(
            needs_layout_passes=False, use_tc_tiling_on_sc=False),
        scratch_types=[
            pltpu.VMEM((K,), jnp.int32),
            pltpu.VMEM((K,), jnp.int32),
            pltpu.VMEM((K, D), jnp.float32),
            pltpu.VMEM((K, DP), jnp.float32),
            pltpu.VMEM((K,), jnp.float32),
            pltpu.VMEM((K,), jnp.float32),
            pltpu.VMEM((K,), jnp.float32),
            pltpu.VMEM((K + 16,), jnp.float32),
            pltpu.VMEM_SHARED((NPAD, DP), jnp.float32),
            pltpu.SemaphoreType.DMA,
        ],
    )
    return kfn(z, el, er, m, src, dst)


def kernel(h, edge_index, W1, aL1, aR1, b1, W2, aL2, aR2, b2, Wfc, bfc):
    src = edge_index[0]
    dst = edge_index[1]
    z1, el1, er1, m1 = _prep(h, W1, aL1, aR1)
    p1 = _edge_aggregate(z1, el1, er1, m1, src, dst)
    z2, el2, er2, m2 = _merge_prep(p1, b1, W2, aL2, aR2)
    p2 = _edge_aggregate(z2, el2, er2, m2, src, dst)
    return _final(p2, b2, Wfc, bfc)
